# Initial kernel scaffold; baseline (speedup 1.0000x reference)
import jax
import jax.numpy as jnp
from jax.experimental import pallas as pl


def kernel(x, edge_index, W_in, b_in, W_self1, b_self1, W_neigh1, b_neigh1, W_self2, b_self2, W_neigh2, b_neigh2, enable_rewire):
    raise NotImplementedError("write your pallas kernel here")


# speedup vs baseline: 7.2373x; 7.2373x over previous
import functools

import jax
import jax.numpy as jnp
from jax import lax
from jax.experimental import pallas as pl
from jax.experimental.pallas import tpu as pltpu
from jax.experimental.pallas import tpu_sc as plsc

NC = 2
NS = 16
NW = NC * NS
K = 80
N_PAD = 10240


def _a_body(x, win, bin_, ws1, bs1, wn1, bn1, m1, s1):
    h = jnp.maximum(
        jnp.dot(x[...], win[...], preferred_element_type=jnp.float32)
        + bin_[...], 0.0)
    m1[...] = jnp.dot(h, wn1[...], preferred_element_type=jnp.float32)
    s1[...] = (jnp.dot(h, ws1[...], preferred_element_type=jnp.float32)
               + bs1[...] + bn1[...])


def _c_body(s1, p0, p1, d0, d1, ws2, bs2, wn2, bn2, m2, s2):
    deg = jnp.maximum(d0[...] + d1[...], 1e-12)
    h1 = jnp.maximum(s1[...] + (p0[...] + p1[...]) / deg, 0.0)
    m2[...] = jnp.dot(h1, wn2[...], preferred_element_type=jnp.float32)
    s2[...] = (jnp.dot(h1, ws2[...], preferred_element_type=jnp.float32)
               + bs2[...] + bn2[...])


def _e_body(s2, q0, q1, d0, d1, out):
    deg = jnp.maximum(d0[...] + d1[...], 1e-12)
    out[...] = s2[...] + (q0[...] + q1[...]) / deg


def _full(shape):
    return pl.BlockSpec(shape, lambda i: (0, 0))


def _rows(br, d):
    return pl.BlockSpec((br, d), lambda i: (i, 0))


@functools.cache
def _make_segsum(e_total, n_rows, d, with_deg):
    assert e_total % (NW * K) == 0
    steps = e_total // (NW * K)
    per_tile = N_PAD // NS
    mesh = plsc.VectorSubcoreMesh(core_axis_name="c", subcore_axis_name="s",
                                  num_cores=NC, num_subcores=NS)

    if with_deg:
        out_type = (jax.ShapeDtypeStruct((NC * N_PAD, d), jnp.float32),
                    jax.ShapeDtypeStruct((NC * N_PAD,), jnp.float32))
        scratch = (pltpu.VMEM((steps, K), jnp.int32),
                   pltpu.VMEM((steps, K), jnp.int32),
                   pltpu.VMEM((K, d), jnp.float32),
                   pltpu.VMEM((K,), jnp.float32),
                   pltpu.VMEM_SHARED((N_PAD, d), jnp.float32),
                   pltpu.VMEM_SHARED((N_PAD,), jnp.float32),
                   pltpu.SemaphoreType.DMA)

        @functools.partial(pl.kernel, mesh=mesh, out_type=out_type,
                           scratch_types=scratch)
        def segsum(m_hbm, row_hbm, col_hbm, zrow_hbm, zdeg_hbm, out_hbm,
                   deg_hbm, rowv, colv, buf, onesv, agg_sh, deg_sh, sem):
            c = lax.axis_index("c")
            s = lax.axis_index("s")
            wid = c * NS + s

            pltpu.sync_copy(row_hbm.at[wid], rowv)
            pltpu.sync_copy(col_hbm.at[wid], colv)
            pltpu.sync_copy(zrow_hbm, agg_sh.at[pl.ds(s * per_tile, per_tile)])
            pltpu.sync_copy(zdeg_hbm, deg_sh.at[pl.ds(s * per_tile, per_tile)])
            for j in range(K // 16):
                onesv[pl.ds(j * 16, 16)] = jnp.ones((16,), jnp.float32)
            plsc.subcore_barrier()

            def step(t, carry):
                pltpu.async_copy(m_hbm.at[colv.at[t]], buf, sem).wait()
                pltpu.sync_copy(buf, agg_sh.at[rowv.at[t]], add=True)
                pltpu.sync_copy(onesv, deg_sh.at[rowv.at[t]], add=True)
                return carry

            lax.fori_loop(0, steps, step, 0)
            plsc.subcore_barrier()

            base = c * N_PAD + s * per_tile
            pltpu.sync_copy(agg_sh.at[pl.ds(s * per_tile, per_tile)],
                            out_hbm.at[pl.ds(base, per_tile)])
            pltpu.sync_copy(deg_sh.at[pl.ds(s * per_tile, per_tile)],
                            deg_hbm.at[pl.ds(base, per_tile)])
    else:
        out_type = jax.ShapeDtypeStruct((NC * N_PAD, d), jnp.float32)
        scratch = (pltpu.VMEM((steps, K), jnp.int32),
                   pltpu.VMEM((steps, K), jnp.int32),
                   pltpu.VMEM((K, d), jnp.float32),
                   pltpu.VMEM_SHARED((N_PAD, d), jnp.float32),
                   pltpu.SemaphoreType.DMA)

        @functools.partial(
            pl.kernel, mesh=mesh, out_type=out_type, scratch_types=scratch,
            compiler_params=pltpu.CompilerParams(use_tc_tiling_on_sc=False))
        def segsum(m_hbm, row_hbm, col_hbm, zrow_hbm, out_hbm,
                   rowv, colv, buf, agg_sh, sem):
            c = lax.axis_index("c")
            s = lax.axis_index("s")
            wid = c * NS + s

            pltpu.sync_copy(row_hbm.at[wid], rowv)
            pltpu.sync_copy(col_hbm.at[wid], colv)
            pltpu.sync_copy(zrow_hbm, agg_sh.at[pl.ds(s * per_tile, per_tile)])
            plsc.subcore_barrier()

            def step(t, carry):
                pltpu.async_copy(m_hbm.at[colv.at[t]], buf, sem).wait()
                pltpu.sync_copy(buf, agg_sh.at[rowv.at[t]], add=True)
                return carry

            lax.fori_loop(0, steps, step, 0)
            plsc.subcore_barrier()

            base = c * N_PAD + s * per_tile
            pltpu.sync_copy(agg_sh.at[pl.ds(s * per_tile, per_tile)],
                            out_hbm.at[pl.ds(base, per_tile)])

    return segsum


def kernel(x, edge_index, W_in, b_in, W_self1, b_self1, W_neigh1, b_neigh1,
           W_self2, b_self2, W_neigh2, b_neigh2, enable_rewire=False):
    n, d_in = x.shape
    d_h = W_in.shape[1]
    d_out = W_self2.shape[1]
    e_total = edge_index.shape[1]
    br = 1000
    grid = (n // br,)

    steps = e_total // (NW * K)
    row2 = edge_index[0].reshape(NW, steps, K)
    col2 = edge_index[1].reshape(NW, steps, K)
    zrow_h = jnp.zeros((N_PAD // NS, d_h), jnp.float32)
    zrow_o = jnp.zeros((N_PAD // NS, d_out), jnp.float32)
    zdeg = jnp.zeros((N_PAD // NS,), jnp.float32)

    b_in2 = b_in.reshape(1, d_h)
    bs1 = b_self1.reshape(1, d_h)
    bn1 = b_neigh1.reshape(1, d_h)
    bs2 = b_self2.reshape(1, d_out)
    bn2 = b_neigh2.reshape(1, d_out)

    m1, s1 = pl.pallas_call(
        _a_body,
        grid=grid,
        in_specs=[_rows(br, d_in), _full((d_in, d_h)), _full((1, d_h)),
                  _full((d_h, d_h)), _full((1, d_h)),
                  _full((d_h, d_h)), _full((1, d_h))],
        out_specs=[_rows(br, d_h), _rows(br, d_h)],
        out_shape=[jax.ShapeDtypeStruct((n, d_h), jnp.float32),
                   jax.ShapeDtypeStruct((n, d_h), jnp.float32)],
    )(x, W_in, b_in2, W_self1, bs1, W_neigh1, bn1)

    segsum1 = _make_segsum(e_total, n, d_h, True)
    agg1, deg = segsum1(m1, row2, col2, zrow_h, zdeg)
    p0 = agg1[:n]
    p1 = agg1[N_PAD:N_PAD + n]
    d0 = deg[:n].reshape(n, 1)
    d1 = deg[N_PAD:N_PAD + n].reshape(n, 1)

    m2, s2 = pl.pallas_call(
        _c_body,
        grid=grid,
        in_specs=[_rows(br, d_h), _rows(br, d_h), _rows(br, d_h),
                  _rows(br, 1), _rows(br, 1),
                  _full((d_h, d_out)), _full((1, d_out)),
                  _full((d_h, d_out)), _full((1, d_out))],
        out_specs=[_rows(br, d_out), _rows(br, d_out)],
        out_shape=[jax.ShapeDtypeStruct((n, d_out), jnp.float32),
                   jax.ShapeDtypeStruct((n, d_out), jnp.float32)],
    )(s1, p0, p1, d0, d1, W_self2, bs2, W_neigh2, bn2)

    segsum2 = _make_segsum(e_total, n, d_out, False)
    agg2 = segsum2(m2, row2, col2, zrow_o)
    q0 = agg2[:n]
    q1 = agg2[N_PAD:N_PAD + n]

    (logits,) = pl.pallas_call(
        _e_body,
        grid=grid,
        in_specs=[_rows(br, d_out), _rows(br, d_out), _rows(br, d_out),
                  _rows(br, 1), _rows(br, 1)],
        out_specs=[_rows(br, d_out)],
        out_shape=[jax.ShapeDtypeStruct((n, d_out), jnp.float32)],
    )(s2, q0, q1, d0, d1)

    return logits
